# baseline (speedup 1.0000x reference)
import functools

import jax
import jax.numpy as jnp
from jax import lax
from jax.experimental import pallas as pl
from jax.experimental.pallas import tpu as pltpu
from jax.experimental.pallas import tpu_sc as plsc

B = 16384
D = 128
V = 1000
LANES = 16
NC = 2
NS = 16
NW = NC * NS
BPW = B // NW
CH = 128
NCH = BPW // CH
NBUF = 3
TROWS = 64
TLAST = V - (NS - 1) * TROWS


def _make_sc_kernel():
    mesh = plsc.VectorSubcoreMesh(
        core_axis_name="c", subcore_axis_name="s", num_cores=NC)

    @functools.partial(
        pl.kernel,
        out_type=jax.ShapeDtypeStruct((B, D), jnp.float32),
        mesh=mesh,
        scratch_types=[
            pltpu.VMEM_SHARED((V, D), jnp.float32),
            pltpu.VMEM((BPW,), jnp.int32),
            pltpu.VMEM((NBUF, CH, D), jnp.float32),
            pltpu.VMEM((NBUF, CH, D), jnp.float32),
            pltpu.SemaphoreType.DMA,
            pltpu.SemaphoreType.DMA,
            pltpu.SemaphoreType.DMA,
            pltpu.SemaphoreType.DMA,
            pltpu.SemaphoreType.DMA,
            pltpu.SemaphoreType.DMA,
            pltpu.SemaphoreType.DMA,
            pltpu.SemaphoreType.DMA,
            pltpu.SemaphoreType.DMA,
            pltpu.SemaphoreType.DMA,
            pltpu.SemaphoreType.DMA,
        ],
    )
    def k(z_hbm, label_hbm, table_hbm, out_hbm,
          table_sh, idx_v, rows_v, z_v,
          gsem0, gsem1, gsem2, zsem0, zsem1, zsem2,
          osem0, osem1, osem2, tsem, lsem):
        cid = lax.axis_index("c")
        sid = lax.axis_index("s")
        wid = sid * NC + cid
        base = wid * BPW
        gsem = (gsem0, gsem1, gsem2)
        zsem = (zsem0, zsem1, zsem2)
        osem = (osem0, osem1, osem2)

        lcp = pltpu.async_copy(label_hbm.at[pl.ds(base, BPW)], idx_v, lsem)

        def start_z(j, b):
            return pltpu.async_copy(
                z_hbm.at[pl.ds(base + j * CH, CH)], z_v.at[b], zsem[b])

        zc0 = start_z(0, 0)
        zc1 = start_z(1, 1)

        @pl.when(sid < NS - 1)
        def _():
            trow = sid * TROWS
            pltpu.async_copy(
                table_hbm.at[pl.ds(trow, TROWS)],
                table_sh.at[pl.ds(trow, TROWS)], tsem).wait()

        @pl.when(sid == NS - 1)
        def _():
            pltpu.async_copy(
                table_hbm.at[pl.ds((NS - 1) * TROWS, TLAST)],
                table_sh.at[pl.ds((NS - 1) * TROWS, TLAST)], tsem).wait()

        lcp.wait()
        plsc.subcore_barrier()

        def start_gather(j, b):
            return pltpu.async_copy(
                table_sh.at[idx_v.at[pl.ds(j * CH, CH)]], rows_v.at[b],
                gsem[b])

        in_flight = [
            (start_gather(0, 0), zc0),
            (start_gather(1, 1), zc1),
        ]
        stores = [None, None, None]
        for j in range(NCH):
            b = j % NBUF
            g, zc = in_flight.pop(0)
            g.wait()
            zc.wait()
            jn = j + 2
            if jn < NCH:
                nb = jn % NBUF
                if stores[nb] is not None:
                    stores[nb].wait()
                    stores[nb] = None
                in_flight.append((start_gather(jn, nb), start_z(jn, nb)))

            rows_b = rows_v.at[b]
            zv_b = z_v.at[b]

            @plsc.parallel_loop(0, CH, unroll=8)
            def mul_row(r):
                for kk in range(D // LANES):
                    s = pl.ds(kk * LANES, LANES)
                    rows_b[r, s] = rows_b[r, s] * zv_b[r, s]

            stores[b] = pltpu.async_copy(
                rows_b, out_hbm.at[pl.ds(base + j * CH, CH)], osem[b])
        for st in stores:
            if st is not None:
                st.wait()

    return k


_sc_kernel = _make_sc_kernel()


def kernel(z, label, emb_table):
    return _sc_kernel(z, label, emb_table)


# speedup vs baseline: 1.0568x; 1.0568x over previous
import functools

import jax
import jax.numpy as jnp
from jax import lax
from jax.experimental import pallas as pl
from jax.experimental.pallas import tpu as pltpu
from jax.experimental.pallas import tpu_sc as plsc

B = 16384
D = 128
V = 1000
LANES = 16
NC = 2
NS = 16
NW = NC * NS
BPW = B // NW
CH = 128
NCH = BPW // CH
NBUF = 3
TROWS = 64
TLAST = V - (NS - 1) * TROWS


def _make_sc_kernel():
    mesh = plsc.VectorSubcoreMesh(
        core_axis_name="c", subcore_axis_name="s", num_cores=NC)

    @functools.partial(
        pl.kernel,
        out_type=jax.ShapeDtypeStruct((B, D), jnp.float32),
        mesh=mesh,
        scratch_types=[
            pltpu.VMEM_SHARED((V, D), jnp.float32),
            pltpu.VMEM((BPW,), jnp.int32),
            pltpu.VMEM((NBUF, CH, D), jnp.float32),
            pltpu.VMEM((NBUF, CH, D), jnp.float32),
            pltpu.SemaphoreType.DMA,
            pltpu.SemaphoreType.DMA,
            pltpu.SemaphoreType.DMA,
            pltpu.SemaphoreType.DMA,
            pltpu.SemaphoreType.DMA,
            pltpu.SemaphoreType.DMA,
            pltpu.SemaphoreType.DMA,
            pltpu.SemaphoreType.DMA,
            pltpu.SemaphoreType.DMA,
            pltpu.SemaphoreType.DMA,
            pltpu.SemaphoreType.DMA,
        ],
    )
    def k(z_hbm, label_hbm, table_hbm, out_hbm,
          table_sh, idx_v, rows_v, z_v,
          gsem0, gsem1, gsem2, zsem0, zsem1, zsem2,
          osem0, osem1, osem2, tsem, lsem):
        cid = lax.axis_index("c")
        sid = lax.axis_index("s")
        wid = sid * NC + cid
        base = wid * BPW
        gsem = (gsem0, gsem1, gsem2)
        zsem = (zsem0, zsem1, zsem2)
        osem = (osem0, osem1, osem2)

        lcp = pltpu.async_copy(label_hbm.at[pl.ds(base, BPW)], idx_v, lsem)

        def start_z(j, b):
            return pltpu.async_copy(
                z_hbm.at[pl.ds(base + j * CH, CH)], z_v.at[b], zsem[b])

        zc0 = start_z(0, 0)
        zc1 = start_z(1, 1)

        @pl.when(sid < NS - 1)
        def _():
            trow = sid * TROWS
            pltpu.async_copy(
                table_hbm.at[pl.ds(trow, TROWS)],
                table_sh.at[pl.ds(trow, TROWS)], tsem).wait()

        @pl.when(sid == NS - 1)
        def _():
            pltpu.async_copy(
                table_hbm.at[pl.ds((NS - 1) * TROWS, TLAST)],
                table_sh.at[pl.ds((NS - 1) * TROWS, TLAST)], tsem).wait()

        lcp.wait()
        plsc.subcore_barrier()

        def start_gather(j, b):
            return pltpu.async_copy(
                table_sh.at[idx_v.at[pl.ds(j * CH, CH)]], rows_v.at[b],
                gsem[b])

        in_flight = [
            (start_gather(0, 0), zc0),
            (start_gather(1, 1), zc1),
        ]
        stores = [None, None, None]
        for j in range(NCH):
            b = j % NBUF
            g, zc = in_flight.pop(0)
            g.wait()
            zc.wait()
            jn = j + 2
            if jn < NCH:
                nb = jn % NBUF
                if stores[nb] is not None:
                    stores[nb].wait()
                    stores[nb] = None
                in_flight.append((start_gather(jn, nb), start_z(jn, nb)))

            rows_b = rows_v.at[b]
            zv_b = z_v.at[b]

            @plsc.parallel_loop(0, CH, unroll=2)
            def mul_row(r):
                for kk in range(D // LANES):
                    s = pl.ds(kk * LANES, LANES)
                    rows_b[r, s] = rows_b[r, s] * zv_b[r, s]

            stores[b] = pltpu.async_copy(
                rows_b, out_hbm.at[pl.ds(base + j * CH, CH)], osem[b])
        for st in stores:
            if st is not None:
                st.wait()

    return k


_sc_kernel = _make_sc_kernel()


def kernel(z, label, emb_table):
    return _sc_kernel(z, label, emb_table)


# speedup vs baseline: 1.0682x; 1.0108x over previous
import functools

import jax
import jax.numpy as jnp
from jax import lax
from jax.experimental import pallas as pl
from jax.experimental.pallas import tpu as pltpu
from jax.experimental.pallas import tpu_sc as plsc

B = 16384
D = 128
V = 1000
LANES = 16
NC = 2
NS = 16
NW = NC * NS
BPW = B // NW
CH = 128
NCH = BPW // CH
NBUF = 3
TROWS = 64
TLAST = V - (NS - 1) * TROWS


def _make_sc_kernel():
    mesh = plsc.VectorSubcoreMesh(
        core_axis_name="c", subcore_axis_name="s", num_cores=NC)

    @functools.partial(
        pl.kernel,
        out_type=jax.ShapeDtypeStruct((B, D), jnp.float32),
        mesh=mesh,
        scratch_types=[
            pltpu.VMEM_SHARED((V, D), jnp.float32),
            pltpu.VMEM((BPW,), jnp.int32),
            pltpu.VMEM((NBUF, CH, D), jnp.float32),
            pltpu.VMEM((NBUF, CH, D), jnp.float32),
            pltpu.SemaphoreType.DMA,
            pltpu.SemaphoreType.DMA,
            pltpu.SemaphoreType.DMA,
            pltpu.SemaphoreType.DMA,
            pltpu.SemaphoreType.DMA,
            pltpu.SemaphoreType.DMA,
            pltpu.SemaphoreType.DMA,
            pltpu.SemaphoreType.DMA,
            pltpu.SemaphoreType.DMA,
            pltpu.SemaphoreType.DMA,
            pltpu.SemaphoreType.DMA,
        ],
    )
    def k(z_hbm, label_hbm, table_hbm, out_hbm,
          table_sh, idx_v, rows_v, z_v,
          gsem0, gsem1, gsem2, zsem0, zsem1, zsem2,
          osem0, osem1, osem2, tsem, lsem):
        cid = lax.axis_index("c")
        sid = lax.axis_index("s")
        wid = sid * NC + cid
        base = wid * BPW
        gsem = (gsem0, gsem1, gsem2)
        zsem = (zsem0, zsem1, zsem2)
        osem = (osem0, osem1, osem2)

        lcp = pltpu.async_copy(label_hbm.at[pl.ds(base, BPW)], idx_v, lsem)

        def start_z(j, b):
            return pltpu.async_copy(
                z_hbm.at[pl.ds(base + j * CH, CH)], z_v.at[b], zsem[b])

        zc0 = start_z(0, 0)
        zc1 = start_z(1, 1)

        @pl.when(sid < NS - 1)
        def _():
            trow = sid * TROWS
            pltpu.async_copy(
                table_hbm.at[pl.ds(trow, TROWS)],
                table_sh.at[pl.ds(trow, TROWS)], tsem).wait()

        @pl.when(sid == NS - 1)
        def _():
            pltpu.async_copy(
                table_hbm.at[pl.ds((NS - 1) * TROWS, TLAST)],
                table_sh.at[pl.ds((NS - 1) * TROWS, TLAST)], tsem).wait()

        lcp.wait()
        plsc.subcore_barrier()

        def start_gather(j, b):
            return pltpu.async_copy(
                table_sh.at[idx_v.at[pl.ds(j * CH, CH)]], rows_v.at[b],
                gsem[b])

        in_flight = [
            (start_gather(0, 0), zc0),
            (start_gather(1, 1), zc1),
        ]
        stores = [None, None, None]
        for j in range(NCH):
            b = j % NBUF
            g, zc = in_flight.pop(0)
            g.wait()
            zc.wait()
            jn = j + 2
            if jn < NCH:
                nb = jn % NBUF
                if stores[nb] is not None:
                    stores[nb].wait()
                    stores[nb] = None
                in_flight.append((start_gather(jn, nb), start_z(jn, nb)))

            rows_b = rows_v.at[b]
            zv_b = z_v.at[b]

            @plsc.parallel_loop(0, CH, unroll=1)
            def mul_row(r):
                for kk in range(D // LANES):
                    s = pl.ds(kk * LANES, LANES)
                    rows_b[r, s] = rows_b[r, s] * zv_b[r, s]

            stores[b] = pltpu.async_copy(
                rows_b, out_hbm.at[pl.ds(base + j * CH, CH)], osem[b])
        for st in stores:
            if st is not None:
                st.wait()

    return k


_sc_kernel = _make_sc_kernel()


def kernel(z, label, emb_table):
    return _sc_kernel(z, label, emb_table)


# speedup vs baseline: 1.0707x; 1.0024x over previous
import functools

import jax
import jax.numpy as jnp
from jax import lax
from jax.experimental import pallas as pl
from jax.experimental.pallas import tpu as pltpu
from jax.experimental.pallas import tpu_sc as plsc

B = 16384
D = 128
V = 1000
LANES = 16
NC = 2
NS = 16
NW = NC * NS
BPW = B // NW
CH = 128
NCH = BPW // CH
NBUF = 3
TROWS = 64
TLAST = V - (NS - 1) * TROWS


def _make_sc_kernel():
    mesh = plsc.VectorSubcoreMesh(
        core_axis_name="c", subcore_axis_name="s", num_cores=NC)

    @functools.partial(
        pl.kernel,
        out_type=jax.ShapeDtypeStruct((B, D), jnp.float32),
        mesh=mesh,
        scratch_types=[
            pltpu.VMEM_SHARED((V, D), jnp.float32),
            pltpu.VMEM((BPW,), jnp.int32),
            pltpu.VMEM((NBUF, CH, D), jnp.float32),
            pltpu.VMEM((NBUF, CH, D), jnp.float32),
            pltpu.SemaphoreType.DMA,
            pltpu.SemaphoreType.DMA,
            pltpu.SemaphoreType.DMA,
            pltpu.SemaphoreType.DMA,
            pltpu.SemaphoreType.DMA,
            pltpu.SemaphoreType.DMA,
            pltpu.SemaphoreType.DMA,
            pltpu.SemaphoreType.DMA,
            pltpu.SemaphoreType.DMA,
            pltpu.SemaphoreType.DMA,
            pltpu.SemaphoreType.DMA,
        ],
    )
    def k(z_hbm, label_hbm, table_hbm, out_hbm,
          table_sh, idx_v, rows_v, z_v,
          gsem0, gsem1, gsem2, zsem0, zsem1, zsem2,
          osem0, osem1, osem2, tsem, lsem):
        cid = lax.axis_index("c")
        sid = lax.axis_index("s")
        wid = sid * NC + cid
        base = wid * BPW
        gsem = (gsem0, gsem1, gsem2)
        zsem = (zsem0, zsem1, zsem2)
        osem = (osem0, osem1, osem2)

        lcp = pltpu.async_copy(label_hbm.at[pl.ds(base, BPW)], idx_v, lsem)

        def start_z(j, b):
            return pltpu.async_copy(
                z_hbm.at[pl.ds(base + j * CH, CH)], z_v.at[b], zsem[b])

        zc0 = start_z(0, 0)
        zc1 = start_z(1, 1)

        @pl.when(sid < NS - 1)
        def _():
            trow = sid * TROWS
            pltpu.async_copy(
                table_hbm.at[pl.ds(trow, TROWS)],
                table_sh.at[pl.ds(trow, TROWS)], tsem).wait()

        @pl.when(sid == NS - 1)
        def _():
            pltpu.async_copy(
                table_hbm.at[pl.ds((NS - 1) * TROWS, TLAST)],
                table_sh.at[pl.ds((NS - 1) * TROWS, TLAST)], tsem).wait()

        lcp.wait()
        plsc.subcore_barrier()

        def start_gather(j, b):
            return pltpu.async_copy(
                table_sh.at[idx_v.at[pl.ds(j * CH, CH)]], rows_v.at[b],
                gsem[b])

        in_flight = [
            (start_gather(0, 0), zc0),
            (start_gather(1, 1), zc1),
        ]
        stores = [None, None, None]
        for j in range(NCH):
            b = j % NBUF
            g, zc = in_flight.pop(0)
            g.wait()
            zc.wait()
            jn = j + 2
            if jn < NCH:
                nb = jn % NBUF
                if stores[nb] is not None:
                    for st in stores[nb]:
                        st.wait()
                    stores[nb] = None
                in_flight.append((start_gather(jn, nb), start_z(jn, nb)))

            rows_b = rows_v.at[b]
            zv_b = z_v.at[b]
            H = CH // 2

            @plsc.parallel_loop(0, H, unroll=1)
            def mul_row_lo(r):
                for kk in range(D // LANES):
                    s = pl.ds(kk * LANES, LANES)
                    rows_b[r, s] = rows_b[r, s] * zv_b[r, s]

            st_lo = pltpu.async_copy(
                rows_b.at[pl.ds(0, H)],
                out_hbm.at[pl.ds(base + j * CH, H)], osem[b])

            @plsc.parallel_loop(H, CH, unroll=1)
            def mul_row_hi(r):
                for kk in range(D // LANES):
                    s = pl.ds(kk * LANES, LANES)
                    rows_b[r, s] = rows_b[r, s] * zv_b[r, s]

            st_hi = pltpu.async_copy(
                rows_b.at[pl.ds(H, H)],
                out_hbm.at[pl.ds(base + j * CH + H, H)], osem[b])
            stores[b] = (st_lo, st_hi)
        for pair in stores:
            if pair is not None:
                for st in pair:
                    st.wait()

    return k


_sc_kernel = _make_sc_kernel()


def kernel(z, label, emb_table):
    return _sc_kernel(z, label, emb_table)
